# baseline (speedup 1.0000x reference)
import functools

import jax
import jax.numpy as jnp
from jax import lax
from jax.experimental import pallas as pl
from jax.experimental.pallas import tpu as pltpu
from jax.experimental.pallas import tpu_sc as plsc

_EPS = 1e-5


def _sc_detile(tab2d, tail, g, v, d):
    nt = v // 128
    ub = 8
    upg = (nt + ub - 1) // ub
    last_vb = nt - ub
    units = g * upg
    info = plsc.get_sparse_core_info()
    nw = info.num_cores * info.num_subcores
    n_per_w = (units + nw - 1) // nw
    nbuf = 3
    n_batch = (n_per_w + nbuf - 1) // nbuf
    tail_w = (v - nt * 128) * d
    cw = ub * 128
    mesh = plsc.VectorSubcoreMesh(core_axis_name="c", subcore_axis_name="s")

    @functools.partial(
        pl.kernel,
        mesh=mesh,
        out_type=jax.ShapeDtypeStruct((g * v * d,), jnp.float32),
        compiler_params=pltpu.CompilerParams(
            use_tc_tiling_on_sc=True, needs_layout_passes=False),
        scratch_types=[
            pltpu.VMEM((nbuf, d, cw), jnp.float32),
            pltpu.VMEM((nbuf * cw * d,), jnp.float32),
            pltpu.VMEM((tail_w,), jnp.float32),
            pltpu.SemaphoreType.DMA((nbuf,)),
            pltpu.SemaphoreType.DMA((nbuf,)),
        ],
    )
    def detile(tab_hbm, tail_hbm, out_hbm, bufs, stages, tbuf, isem, osem):
        wid = lax.axis_index("s") * info.num_cores + lax.axis_index("c")
        b16 = lax.iota(jnp.int32, 16) * d
        b16d = [b16 + dd for dd in range(d)]

        def unit_slices(u):
            u = jnp.minimum(u, units - 1)
            gi = u // upg
            j = u - gi * upg
            vb0 = jnp.minimum(j * ub, last_vb)
            r0 = pl.multiple_of(gi * d, 8)
            c0 = pl.multiple_of(vb0 * 128, 128)
            e0 = pl.multiple_of((gi * v + vb0 * 128) * d, 128)
            return r0, c0, e0

        def batch_body(kk, carry):
            k0 = kk * nbuf
            ins = []
            for b in range(nbuf):
                r0, c0, _ = unit_slices(wid * n_per_w + k0 + b)
                ins.append(pltpu.async_copy(
                    tab_hbm.at[pl.ds(r0, d), pl.ds(c0, cw)],
                    bufs.at[b], isem.at[b]))
            outs = []
            for b in range(nbuf):
                _, _, e0 = unit_slices(wid * n_per_w + k0 + b)
                ins[b].wait()
                buf = bufs.at[b]
                stage = stages.at[pl.ds(b * cw * d, cw * d)]

                @plsc.parallel_loop(0, cw // 16, unroll=8)
                def _shuffle(c, buf=buf, stage=stage):
                    c0 = c * (16 * d)
                    for dd in range(d):
                        vals = buf[dd, pl.ds(c * 16, 16)]
                        plsc.store_scatter(stage, [b16d[dd] + c0], vals)
                outs.append(pltpu.async_copy(
                    stage, out_hbm.at[pl.ds(e0, cw * d)], osem.at[b]))
            for b in range(nbuf):
                outs[b].wait()
            return carry

        lax.fori_loop(0, n_batch, batch_body, 0)

        @pl.when(wid < g)
        def _():
            pltpu.sync_copy(tail_hbm.at[wid], tbuf)
            e0 = pl.multiple_of(wid * v * d + nt * 128 * d, 128)
            pltpu.sync_copy(tbuf, out_hbm.at[pl.ds(e0, tail_w)])

    return detile(tab2d, tail)


def _sc_gather(flat_idx, tables2d):
    n = flat_idx.shape[0]
    d = tables2d.shape[1]
    info = plsc.get_sparse_core_info()
    nw = info.num_cores * info.num_subcores
    per_w = n // nw
    ch = 1664
    n_ch = per_w // ch
    assert per_w % ch == 0

    mesh = plsc.VectorSubcoreMesh(core_axis_name="c", subcore_axis_name="s")

    @functools.partial(
        pl.kernel,
        mesh=mesh,
        out_type=jax.ShapeDtypeStruct((n, d), jnp.float32),
        compiler_params=pltpu.CompilerParams(use_tc_tiling_on_sc=False),
        scratch_types=[
            pltpu.VMEM((ch,), jnp.int32),
            pltpu.VMEM((ch, d), jnp.float32),
            pltpu.SemaphoreType.DMA,
        ],
    )
    def gather_kernel(idx_hbm, tab_hbm, out_hbm, idx_v, rows_v, sem):
        wid = lax.axis_index("s") * info.num_cores + lax.axis_index("c")
        base = wid * per_w

        def body(i, carry):
            off = base + i * ch
            pltpu.sync_copy(idx_hbm.at[pl.ds(off, ch)], idx_v)
            pltpu.async_copy(tab_hbm.at[idx_v], rows_v, sem).wait()
            pltpu.sync_copy(rows_v, out_hbm.at[pl.ds(off, ch)])
            return carry

        lax.fori_loop(0, n_ch, body, 0)

    return gather_kernel(flat_idx, tables2d)


def _mm_stats_body(x_ref, w_ref, b_ref, h_ref, s_ref, q_ref):
    j = pl.program_id(0)
    h = jnp.dot(x_ref[...], w_ref[...], preferred_element_type=jnp.float32)
    h = h + b_ref[...]
    h_ref[...] = h

    @pl.when(j == 0)
    def _():
        s_ref[...] = jnp.zeros_like(s_ref)
        q_ref[...] = jnp.zeros_like(q_ref)

    s_ref[...] += jnp.sum(h, axis=0, keepdims=True)
    q_ref[...] += jnp.sum(h * h, axis=0, keepdims=True)


def _bn_affine(s, q, ga, bea, gb, beb, nb):
    m = s / nb
    v = q / nb - m * m
    inv1 = lax.rsqrt(v + _EPS)
    sa = ga * inv1
    v2 = sa * sa * v
    inv2 = lax.rsqrt(v2 + _EPS)
    scale = sa * gb * inv2
    return m, scale


def _norm_mm_stats_body(h_ref, s_in, q_in, ga, bea, gb, beb, w_ref, b_ref,
                        h2_ref, s_ref, q_ref, *, nb):
    j = pl.program_id(0)
    m, scale = _bn_affine(s_in[...], q_in[...], ga[...], bea[...],
                          gb[...], beb[...], nb)
    z = jnp.maximum((h_ref[...] - m) * scale + beb[...], 0.0)
    h2 = jnp.dot(z, w_ref[...], preferred_element_type=jnp.float32)
    h2 = h2 + b_ref[...]
    h2_ref[...] = h2

    @pl.when(j == 0)
    def _():
        s_ref[...] = jnp.zeros_like(s_ref)
        q_ref[...] = jnp.zeros_like(q_ref)

    s_ref[...] += jnp.sum(h2, axis=0, keepdims=True)
    q_ref[...] += jnp.sum(h2 * h2, axis=0, keepdims=True)


def _norm_out_body(h_ref, s_in, q_in, ga, bea, gb, beb, w_ref, b_ref,
                   o_ref, *, nb):
    m, scale = _bn_affine(s_in[...], q_in[...], ga[...], bea[...],
                          gb[...], beb[...], nb)
    z = jnp.maximum((h_ref[...] - m) * scale + beb[...], 0.0)
    o = jnp.dot(z, w_ref[...], preferred_element_type=jnp.float32)
    o_ref[...] = jax.nn.sigmoid(o + b_ref[...])


def _row2(a):
    return a.reshape(1, -1)


def kernel(idx, tables, W1, b1, g1a, be1a, g1b, be1b, W2, b2, g2a, be2a,
           g2b, be2b, W3, b3):
    bsz, g = idx.shape
    _, v, d = tables.shape
    gd, h1d = W1.shape
    h2d = W2.shape[1]

    offs = (jnp.arange(g, dtype=jnp.int32) * v)[None, :]
    flat_idx = (idx.astype(jnp.int32) + offs).reshape(-1)
    tab2d = tables.transpose(0, 2, 1).reshape(g * d, v)
    nt = v // 128
    tail = tables[:, nt * 128:, :].reshape(g, (v - nt * 128) * d)
    packed = _sc_detile(tab2d, tail, g, v, d)
    rows = _sc_gather(flat_idx, packed.reshape(g * v, d))
    x = rows.reshape(bsz, gd)

    r = 2048
    nblk = bsz // r
    fullspec = lambda shp: pl.BlockSpec(shp, lambda j: (0, 0))

    h1, s1, q1 = pl.pallas_call(
        _mm_stats_body,
        grid=(nblk,),
        in_specs=[
            pl.BlockSpec((r, gd), lambda j: (j, 0)),
            fullspec((gd, h1d)),
            fullspec((1, h1d)),
        ],
        out_specs=[
            pl.BlockSpec((r, h1d), lambda j: (j, 0)),
            fullspec((1, h1d)),
            fullspec((1, h1d)),
        ],
        out_shape=[
            jax.ShapeDtypeStruct((bsz, h1d), jnp.float32),
            jax.ShapeDtypeStruct((1, h1d), jnp.float32),
            jax.ShapeDtypeStruct((1, h1d), jnp.float32),
        ],
    )(x, W1, _row2(b1))

    h2, s2, q2 = pl.pallas_call(
        functools.partial(_norm_mm_stats_body, nb=float(bsz)),
        grid=(nblk,),
        in_specs=[
            pl.BlockSpec((r, h1d), lambda j: (j, 0)),
            fullspec((1, h1d)), fullspec((1, h1d)),
            fullspec((1, h1d)), fullspec((1, h1d)),
            fullspec((1, h1d)), fullspec((1, h1d)),
            fullspec((h1d, h2d)),
            fullspec((1, h2d)),
        ],
        out_specs=[
            pl.BlockSpec((r, h2d), lambda j: (j, 0)),
            fullspec((1, h2d)),
            fullspec((1, h2d)),
        ],
        out_shape=[
            jax.ShapeDtypeStruct((bsz, h2d), jnp.float32),
            jax.ShapeDtypeStruct((1, h2d), jnp.float32),
            jax.ShapeDtypeStruct((1, h2d), jnp.float32),
        ],
    )(h1, s1, q1, _row2(g1a), _row2(be1a), _row2(g1b), _row2(be1b),
      W2, _row2(b2))

    out = pl.pallas_call(
        functools.partial(_norm_out_body, nb=float(bsz)),
        grid=(nblk,),
        in_specs=[
            pl.BlockSpec((r, h2d), lambda j: (j, 0)),
            fullspec((1, h2d)), fullspec((1, h2d)),
            fullspec((1, h2d)), fullspec((1, h2d)),
            fullspec((1, h2d)), fullspec((1, h2d)),
            fullspec((h2d, 1)),
            fullspec((1, 1)),
        ],
        out_specs=pl.BlockSpec((r, 1), lambda j: (j, 0)),
        out_shape=jax.ShapeDtypeStruct((bsz, 1), jnp.float32),
    )(h2, s2, q2, _row2(g2a), _row2(be2a), _row2(g2b), _row2(be2b),
      W3, _row2(b3))

    return out


# speedup vs baseline: 1.2015x; 1.2015x over previous
import functools

import jax
import jax.numpy as jnp
from jax import lax
from jax.experimental import pallas as pl
from jax.experimental.pallas import tpu as pltpu
from jax.experimental.pallas import tpu_sc as plsc

_EPS = 1e-5


def _sc_detile(tab2d, tail, g, v, d):
    nt = v // 128
    ub = 8
    upg = (nt + ub - 1) // ub
    last_vb = nt - ub
    units = g * upg
    info = plsc.get_sparse_core_info()
    nw = info.num_cores * info.num_subcores
    n_per_w = (units + nw - 1) // nw
    nbuf = 3
    n_batch = (n_per_w + nbuf - 1) // nbuf
    tail_w = (v - nt * 128) * d
    cw = ub * 128
    mesh = plsc.VectorSubcoreMesh(core_axis_name="c", subcore_axis_name="s")

    @functools.partial(
        pl.kernel,
        mesh=mesh,
        out_type=jax.ShapeDtypeStruct((g * v * d,), jnp.float32),
        compiler_params=pltpu.CompilerParams(
            use_tc_tiling_on_sc=True, needs_layout_passes=False),
        scratch_types=[
            pltpu.VMEM((nbuf, d, cw), jnp.float32),
            pltpu.VMEM((nbuf * cw * d,), jnp.float32),
            pltpu.VMEM((tail_w,), jnp.float32),
            pltpu.SemaphoreType.DMA((nbuf,)),
            pltpu.SemaphoreType.DMA((nbuf,)),
        ],
    )
    def detile(tab_hbm, tail_hbm, out_hbm, bufs, stages, tbuf, isem, osem):
        wid = lax.axis_index("s") * info.num_cores + lax.axis_index("c")
        b16 = lax.iota(jnp.int32, 16) * d
        b16d = [b16 + dd for dd in range(d)]

        def unit_slices(u):
            u = jnp.minimum(u, units - 1)
            gi = u // upg
            j = u - gi * upg
            vb0 = jnp.minimum(j * ub, last_vb)
            r0 = pl.multiple_of(gi * d, 8)
            c0 = pl.multiple_of(vb0 * 128, 128)
            e0 = pl.multiple_of((gi * v + vb0 * 128) * d, 128)
            return r0, c0, e0

        def batch_body(kk, carry):
            k0 = kk * nbuf
            ins = []
            for b in range(nbuf):
                r0, c0, _ = unit_slices(wid * n_per_w + k0 + b)
                ins.append(pltpu.async_copy(
                    tab_hbm.at[pl.ds(r0, d), pl.ds(c0, cw)],
                    bufs.at[b], isem.at[b]))
            outs = []
            for b in range(nbuf):
                _, _, e0 = unit_slices(wid * n_per_w + k0 + b)
                ins[b].wait()
                buf = bufs.at[b]
                stage = stages.at[pl.ds(b * cw * d, cw * d)]

                @plsc.parallel_loop(0, cw // 16, unroll=4)
                def _shuffle(c, buf=buf, stage=stage):
                    c0 = c * (16 * d)
                    for dd in range(d):
                        vals = buf[dd, pl.ds(c * 16, 16)]
                        plsc.store_scatter(stage, [b16d[dd] + c0], vals)
                outs.append(pltpu.async_copy(
                    stage, out_hbm.at[pl.ds(e0, cw * d)], osem.at[b]))
            for b in range(nbuf):
                outs[b].wait()
            return carry

        lax.fori_loop(0, n_batch, batch_body, 0)

        @pl.when(wid < g)
        def _():
            pltpu.sync_copy(tail_hbm.at[wid], tbuf)
            e0 = pl.multiple_of(wid * v * d + nt * 128 * d, 128)
            pltpu.sync_copy(tbuf, out_hbm.at[pl.ds(e0, tail_w)])

    return detile(tab2d, tail)


def _sc_gather(flat_idx, tables2d):
    n = flat_idx.shape[0]
    d = tables2d.shape[1]
    info = plsc.get_sparse_core_info()
    nw = info.num_cores * info.num_subcores
    per_w = n // nw
    ch = 1664
    n_ch = per_w // ch
    assert per_w % ch == 0

    mesh = plsc.VectorSubcoreMesh(core_axis_name="c", subcore_axis_name="s")

    @functools.partial(
        pl.kernel,
        mesh=mesh,
        out_type=jax.ShapeDtypeStruct((n, d), jnp.float32),
        compiler_params=pltpu.CompilerParams(use_tc_tiling_on_sc=False),
        scratch_types=[
            pltpu.VMEM((ch,), jnp.int32),
            pltpu.VMEM((ch, d), jnp.float32),
            pltpu.SemaphoreType.DMA,
        ],
    )
    def gather_kernel(idx_hbm, tab_hbm, out_hbm, idx_v, rows_v, sem):
        wid = lax.axis_index("s") * info.num_cores + lax.axis_index("c")
        base = wid * per_w

        def body(i, carry):
            off = base + i * ch
            pltpu.sync_copy(idx_hbm.at[pl.ds(off, ch)], idx_v)
            pltpu.async_copy(tab_hbm.at[idx_v], rows_v, sem).wait()
            pltpu.sync_copy(rows_v, out_hbm.at[pl.ds(off, ch)])
            return carry

        lax.fori_loop(0, n_ch, body, 0)

    return gather_kernel(flat_idx, tables2d)


def _mm_stats_body(x_ref, w_ref, b_ref, h_ref, s_ref, q_ref):
    j = pl.program_id(0)
    h = jnp.dot(x_ref[...], w_ref[...], preferred_element_type=jnp.float32)
    h = h + b_ref[...]
    h_ref[...] = h

    @pl.when(j == 0)
    def _():
        s_ref[...] = jnp.zeros_like(s_ref)
        q_ref[...] = jnp.zeros_like(q_ref)

    s_ref[...] += jnp.sum(h, axis=0, keepdims=True)
    q_ref[...] += jnp.sum(h * h, axis=0, keepdims=True)


def _bn_affine(s, q, ga, bea, gb, beb, nb):
    m = s / nb
    v = q / nb - m * m
    inv1 = lax.rsqrt(v + _EPS)
    sa = ga * inv1
    v2 = sa * sa * v
    inv2 = lax.rsqrt(v2 + _EPS)
    scale = sa * gb * inv2
    return m, scale


def _norm_mm_stats_body(h_ref, s_in, q_in, ga, bea, gb, beb, w_ref, b_ref,
                        h2_ref, s_ref, q_ref, *, nb):
    j = pl.program_id(0)
    m, scale = _bn_affine(s_in[...], q_in[...], ga[...], bea[...],
                          gb[...], beb[...], nb)
    z = jnp.maximum((h_ref[...] - m) * scale + beb[...], 0.0)
    h2 = jnp.dot(z, w_ref[...], preferred_element_type=jnp.float32)
    h2 = h2 + b_ref[...]
    h2_ref[...] = h2

    @pl.when(j == 0)
    def _():
        s_ref[...] = jnp.zeros_like(s_ref)
        q_ref[...] = jnp.zeros_like(q_ref)

    s_ref[...] += jnp.sum(h2, axis=0, keepdims=True)
    q_ref[...] += jnp.sum(h2 * h2, axis=0, keepdims=True)


def _norm_out_body(h_ref, s_in, q_in, ga, bea, gb, beb, w_ref, b_ref,
                   o_ref, *, nb):
    m, scale = _bn_affine(s_in[...], q_in[...], ga[...], bea[...],
                          gb[...], beb[...], nb)
    z = jnp.maximum((h_ref[...] - m) * scale + beb[...], 0.0)
    o = jnp.dot(z, w_ref[...], preferred_element_type=jnp.float32)
    o_ref[...] = jax.nn.sigmoid(o + b_ref[...])


def _row2(a):
    return a.reshape(1, -1)


def kernel(idx, tables, W1, b1, g1a, be1a, g1b, be1b, W2, b2, g2a, be2a,
           g2b, be2b, W3, b3):
    bsz, g = idx.shape
    _, v, d = tables.shape
    gd, h1d = W1.shape
    h2d = W2.shape[1]

    offs = (jnp.arange(g, dtype=jnp.int32) * v)[None, :]
    flat_idx = (idx.astype(jnp.int32) + offs).reshape(-1)
    tab2d = tables.transpose(0, 2, 1).reshape(g * d, v)
    nt = v // 128
    tail = tables[:, nt * 128:, :].reshape(g, (v - nt * 128) * d)
    packed = _sc_detile(tab2d, tail, g, v, d)
    rows = _sc_gather(flat_idx, packed.reshape(g * v, d))
    x = rows.reshape(bsz, gd)

    r = 2048
    nblk = bsz // r
    fullspec = lambda shp: pl.BlockSpec(shp, lambda j: (0, 0))

    h1, s1, q1 = pl.pallas_call(
        _mm_stats_body,
        grid=(nblk,),
        in_specs=[
            pl.BlockSpec((r, gd), lambda j: (j, 0)),
            fullspec((gd, h1d)),
            fullspec((1, h1d)),
        ],
        out_specs=[
            pl.BlockSpec((r, h1d), lambda j: (j, 0)),
            fullspec((1, h1d)),
            fullspec((1, h1d)),
        ],
        out_shape=[
            jax.ShapeDtypeStruct((bsz, h1d), jnp.float32),
            jax.ShapeDtypeStruct((1, h1d), jnp.float32),
            jax.ShapeDtypeStruct((1, h1d), jnp.float32),
        ],
    )(x, W1, _row2(b1))

    h2, s2, q2 = pl.pallas_call(
        functools.partial(_norm_mm_stats_body, nb=float(bsz)),
        grid=(nblk,),
        in_specs=[
            pl.BlockSpec((r, h1d), lambda j: (j, 0)),
            fullspec((1, h1d)), fullspec((1, h1d)),
            fullspec((1, h1d)), fullspec((1, h1d)),
            fullspec((1, h1d)), fullspec((1, h1d)),
            fullspec((h1d, h2d)),
            fullspec((1, h2d)),
        ],
        out_specs=[
            pl.BlockSpec((r, h2d), lambda j: (j, 0)),
            fullspec((1, h2d)),
            fullspec((1, h2d)),
        ],
        out_shape=[
            jax.ShapeDtypeStruct((bsz, h2d), jnp.float32),
            jax.ShapeDtypeStruct((1, h2d), jnp.float32),
            jax.ShapeDtypeStruct((1, h2d), jnp.float32),
        ],
    )(h1, s1, q1, _row2(g1a), _row2(be1a), _row2(g1b), _row2(be1b),
      W2, _row2(b2))

    out = pl.pallas_call(
        functools.partial(_norm_out_body, nb=float(bsz)),
        grid=(nblk,),
        in_specs=[
            pl.BlockSpec((r, h2d), lambda j: (j, 0)),
            fullspec((1, h2d)), fullspec((1, h2d)),
            fullspec((1, h2d)), fullspec((1, h2d)),
            fullspec((1, h2d)), fullspec((1, h2d)),
            fullspec((h2d, 1)),
            fullspec((1, 1)),
        ],
        out_specs=pl.BlockSpec((r, 1), lambda j: (j, 0)),
        out_shape=jax.ShapeDtypeStruct((bsz, 1), jnp.float32),
    )(h2, s2, q2, _row2(g2a), _row2(be2a), _row2(g2b), _row2(be2b),
      W3, _row2(b3))

    return out
